# baseline (device time: 85361 ns/iter reference)
import jax
import jax.numpy as jnp
from jax import lax
from jax.experimental import pallas as pl
from jax.experimental.pallas import tpu as pltpu

N_DEV = 32
N_TOK = 1024
D = 512
H = 1024
E_LOCAL = 4
ROWS = N_TOK // N_DEV


def kernel(x, router_W, route_idx, expert_W, shared_W):
    def body(x_ref, rw_ref, idx_ref, ew_ref, sw_ref, out_ref,
             partial_ref, recv_ref, send_sems, recv_sems, ack_sem):
        my = lax.axis_index("i")

        scores = jnp.dot(x_ref[:, :], rw_ref[:, :],
                         preferred_element_type=jnp.float32)
        smax = jnp.max(scores, axis=1, keepdims=True)
        ex = jnp.exp(scores - smax)
        denom = jnp.sum(ex, axis=1, keepdims=True)
        idx = idx_ref[:, :]
        cols = lax.broadcasted_iota(jnp.int32, scores.shape, 1)
        picked = jnp.sum(jnp.where(cols == idx, ex, 0.0), axis=1,
                         keepdims=True)
        p = picked / denom

        acc = jnp.zeros((N_TOK, H), jnp.float32)
        for j in range(E_LOCAL):
            gid = my * E_LOCAL + j
            w = jnp.where(idx == gid, p, 0.0)
            a = x_ref[:, :] * w
            acc = acc + jnp.dot(a, ew_ref[j],
                                preferred_element_type=jnp.float32)
        partial_ref[:, :] = acc

        rdmas = []
        for k in range(1, N_DEV):
            t = lax.rem(my + k, N_DEV)
            rdma = pltpu.make_async_remote_copy(
                src_ref=partial_ref.at[pl.ds(t * ROWS, ROWS), :],
                dst_ref=recv_ref.at[k],
                send_sem=send_sems.at[k],
                recv_sem=recv_sems.at[k],
                device_id=(t,),
                device_id_type=pl.DeviceIdType.MESH,
            )
            rdma.start()
            rdmas.append(rdma)

        x_own = x_ref[pl.ds(my * ROWS, ROWS), :]
        shared = jnp.dot(x_own, sw_ref[:, :],
                         preferred_element_type=jnp.float32)

        for r in rdmas:
            r.wait_recv()
        remote = jnp.sum(recv_ref[1:N_DEV], axis=0)
        own = partial_ref[pl.ds(my * ROWS, ROWS), :]
        out_ref[:, :] = shared + own + remote

        for r in rdmas:
            r.wait_send()

        for k in range(1, N_DEV):
            s = lax.rem(my - k + N_DEV, N_DEV)
            pl.semaphore_signal(ack_sem, inc=1, device_id=(s,),
                                device_id_type=pl.DeviceIdType.MESH)
        pl.semaphore_wait(ack_sem, N_DEV - 1)

    return pl.pallas_call(
        body,
        out_shape=jax.ShapeDtypeStruct((ROWS, H), jnp.float32),
        in_specs=[pl.BlockSpec(memory_space=pltpu.VMEM)] * 5,
        out_specs=pl.BlockSpec(memory_space=pltpu.VMEM),
        scratch_shapes=[
            pltpu.VMEM((N_TOK, H), jnp.float32),
            pltpu.VMEM((N_DEV, ROWS, H), jnp.float32),
            pltpu.SemaphoreType.DMA((N_DEV,)),
            pltpu.SemaphoreType.DMA((N_DEV,)),
            pltpu.SemaphoreType.REGULAR,
        ],
    )(x, router_W, route_idx, expert_W, shared_W)


# device time: 60270 ns/iter; 1.4163x vs baseline; 1.4163x over previous
import jax
import jax.numpy as jnp
from jax import lax
from jax.experimental import pallas as pl
from jax.experimental.pallas import tpu as pltpu

N_DEV = 32
N_TOK = 1024
D = 512
H = 1024
E_LOCAL = 4
ROWS = N_TOK // N_DEV


def kernel(x, router_W, route_idx, expert_W, shared_W):
    def body(x_ref, rw_ref, idx_ref, ew_ref, sw_ref, out_ref,
             partial_ref, recv_ref, send_sems, recv_sems, ack_sem):
        my = lax.axis_index("i")

        scores = jnp.dot(x_ref[:, :], rw_ref[:, :],
                         preferred_element_type=jnp.float32)
        smax = jnp.max(scores, axis=1, keepdims=True)
        ex = jnp.exp(scores - smax)
        denom = jnp.sum(ex, axis=1, keepdims=True)
        idx = idx_ref[:, :]
        cols = lax.broadcasted_iota(jnp.int32, scores.shape, 1)
        picked = jnp.sum(jnp.where(cols == idx, ex, 0.0), axis=1,
                         keepdims=True)
        p = picked / denom

        acc = jnp.zeros((N_TOK, H), jnp.float32)
        for j in range(E_LOCAL):
            gid = my * E_LOCAL + j
            w = jnp.where(idx == gid, p, 0.0)
            a = (x_ref[:, :] * w).astype(jnp.bfloat16)
            acc = acc + jnp.dot(a, ew_ref[j].astype(jnp.bfloat16),
                                preferred_element_type=jnp.float32)
        partial_ref[:, :] = acc.astype(jnp.bfloat16)

        rdmas = []
        for k in range(1, N_DEV):
            t = lax.rem(my + k, N_DEV)
            rdma = pltpu.make_async_remote_copy(
                src_ref=partial_ref.at[pl.ds(t * ROWS, ROWS), :],
                dst_ref=recv_ref.at[k],
                send_sem=send_sems.at[k],
                recv_sem=recv_sems.at[k],
                device_id=(t,),
                device_id_type=pl.DeviceIdType.MESH,
            )
            rdma.start()
            rdmas.append(rdma)

        x_own = x_ref[pl.ds(my * ROWS, ROWS), :]
        shared = jnp.dot(x_own, sw_ref[:, :],
                         preferred_element_type=jnp.float32)

        for r in rdmas:
            r.wait_recv()
        remote = jnp.sum(recv_ref[1:N_DEV].astype(jnp.float32), axis=0)
        own = partial_ref[pl.ds(my * ROWS, ROWS), :].astype(jnp.float32)
        out_ref[:, :] = shared + own + remote

        for r in rdmas:
            r.wait_send()

        for k in range(1, N_DEV):
            s = lax.rem(my - k + N_DEV, N_DEV)
            pl.semaphore_signal(ack_sem, inc=1, device_id=(s,),
                                device_id_type=pl.DeviceIdType.MESH)
        pl.semaphore_wait(ack_sem, N_DEV - 1)

    return pl.pallas_call(
        body,
        out_shape=jax.ShapeDtypeStruct((ROWS, H), jnp.float32),
        in_specs=[pl.BlockSpec(memory_space=pltpu.VMEM)] * 5,
        out_specs=pl.BlockSpec(memory_space=pltpu.VMEM),
        scratch_shapes=[
            pltpu.VMEM((N_TOK, H), jnp.bfloat16),
            pltpu.VMEM((N_DEV, ROWS, H), jnp.bfloat16),
            pltpu.SemaphoreType.DMA((N_DEV,)),
            pltpu.SemaphoreType.DMA((N_DEV,)),
            pltpu.SemaphoreType.REGULAR,
        ],
    )(x, router_W, route_idx, expert_W, shared_W)
